# baseline (speedup 1.0000x reference)
import jax
import jax.numpy as jnp
from jax import lax
from jax.experimental import pallas as pl
from jax.experimental.pallas import tpu as pltpu
from jax.experimental.pallas import tpu_sc as plsc

B, H, W, C = 4, 512, 512, 96
P = B * H * W
NC, NS, L = 2, 16, 16
NW = NC * NS
PPW = P // NW
CHUNK = 64
NCHUNK = PPW // CHUNK
NVEC = C // L
NBUF = 2


def _body(u_hbm, gx_hbm, gy_hbm, out_hbm, *scr):
    gx_v = scr[0:2]
    gy_v = scr[2:4]
    rows = (scr[4:8], scr[8:12])
    idxs = (scr[12:16], scr[16:20])
    ws = (scr[20:24], scr[24:28])
    out_v = scr[28]
    sem_grid = scr[29:31]
    sem_gat = scr[31:33]
    sem_out = scr[33]

    wid = lax.axis_index("s") * NC + lax.axis_index("c")
    base = wid * PPW
    iota = lax.iota(jnp.int32, L)
    last = NCHUNK - 1

    def grid_start(ci, j):
        cb = base + ci * CHUNK
        pltpu.make_async_copy(gx_hbm.at[pl.ds(cb, CHUNK)], gx_v[j],
                              sem_grid[j]).start()
        pltpu.make_async_copy(gy_hbm.at[pl.ds(cb, CHUNK)], gy_v[j],
                              sem_grid[j]).start()

    def grid_wait(j):
        pltpu.make_async_copy(gx_hbm.at[pl.ds(base, CHUNK)], gx_v[j],
                              sem_grid[j]).wait()
        pltpu.make_async_copy(gy_hbm.at[pl.ds(base, CHUNK)], gy_v[j],
                              sem_grid[j]).wait()

    def compute_idx(ci, j):
        cb = base + ci * CHUNK
        idxa_v, idxb_v, idxc_v, idxd_v = idxs[j]
        wa_v, wb_v, wc_v, wd_v = ws[j]
        for gi in range(CHUNK // L):
            off = gi * L
            gx = gx_v[j][pl.ds(off, L)]
            gy = gy_v[j][pl.ds(off, L)]
            px = 0.5 * ((gx + 1.0) * jnp.float32(W - 1))
            py = 0.5 * ((gy + 1.0) * jnp.float32(H - 1))
            x0 = px.astype(jnp.int32)
            y0 = py.astype(jnp.int32)
            x1 = jnp.minimum(x0 + 1, W - 1)
            y1 = jnp.minimum(y0 + 1, H - 1)
            x0f = x0.astype(jnp.float32)
            y0f = y0.astype(jnp.float32)
            x1f = x1.astype(jnp.float32)
            y1f = y1.astype(jnp.float32)

            p = cb + off + iota
            bb = (p >> 18) << 18
            ra = bb + (y0 << 9) + x0
            rb = bb + (y1 << 9) + x0
            dx01 = x1 - x0
            sl = pl.ds(off, L)
            idxa_v[sl] = ra
            idxb_v[sl] = rb
            idxc_v[sl] = ra + dx01
            idxd_v[sl] = rb + dx01

            dxa = x1f - px
            dxb = px - x0f
            dya = y1f - py
            dyb = py - y0f
            wa_v[sl] = dxa * dya
            wb_v[sl] = dxa * dyb
            wc_v[sl] = dxb * dya
            wd_v[sl] = dxb * dyb

    def gathers_start(j):
        for idx_v, rv in zip(idxs[j], rows[j]):
            pltpu.make_async_copy(u_hbm.at[idx_v], rv, sem_gat[j]).start()

    def gathers_wait(j):
        for idx_v, rv in zip(idxs[j], rows[j]):
            pltpu.make_async_copy(u_hbm.at[idx_v], rv, sem_gat[j]).wait()

    def out_start(ci):
        cb = base + ci * CHUNK
        pltpu.make_async_copy(out_v, out_hbm.at[pl.ds(cb * C, CHUNK * C)],
                              sem_out).start()

    def out_wait():
        pltpu.make_async_copy(out_v, out_hbm.at[pl.ds(base * C, CHUNK * C)],
                              sem_out).wait()

    def combine(j):
        ia_v, ib_v, ic_v, id_v = rows[j]
        wa_v, wb_v, wc_v, wd_v = ws[j]

        for gi in range(CHUNK // L):
            gb = gi * L
            slg = pl.ds(gb, L)
            wav = wa_v[slg]
            wbv = wb_v[slg]
            wcv = wc_v[slg]
            wdv = wd_v[slg]
            for t in range(L):
                pi = gb + t
                wa = wav[t]
                wb = wbv[t]
                wc = wcv[t]
                wd = wdv[t]
                for v in range(NVEC):
                    slv = pl.ds(v * L, L)
                    val = (ia_v[pi, slv] * wa + ib_v[pi, slv] * wb
                           + ic_v[pi, slv] * wc + id_v[pi, slv] * wd)
                    out_v[pl.ds(pi * C + v * L, L)] = val

    grid_start(0, 0)
    grid_wait(0)
    compute_idx(0, 0)
    gathers_start(0)
    grid_start(1, 1)

    def body(k, _):
        c0 = 2 * k
        c1 = c0 + 1
        grid_wait(1)
        compute_idx(c1, 1)
        gathers_start(1)
        grid_start(jnp.minimum(c0 + 2, last), 0)
        gathers_wait(0)

        @pl.when(k > 0)
        def _w0():
            out_wait()

        combine(0)
        out_start(c0)
        grid_wait(0)
        compute_idx(jnp.minimum(c0 + 2, last), 0)
        gathers_start(0)
        grid_start(jnp.minimum(c1 + 2, last), 1)
        gathers_wait(1)
        out_wait()
        combine(1)
        out_start(c1)
        return _

    lax.fori_loop(0, NCHUNK // 2, body, None)

    gathers_wait(0)
    grid_wait(1)
    out_wait()


@jax.jit
def _sample(u2, gx, gy):
    mesh = plsc.VectorSubcoreMesh(core_axis_name="c", subcore_axis_name="s",
                                  num_cores=NC, num_subcores=NS)
    vf = lambda *s: pltpu.VMEM(s, jnp.float32)
    vi = lambda *s: pltpu.VMEM(s, jnp.int32)
    scratch = (
        [vf(CHUNK)] * 2 + [vf(CHUNK)] * 2
        + [vf(CHUNK, C)] * 8
        + [vi(CHUNK)] * 8
        + [vf(CHUNK)] * 8
        + [vf(CHUNK * C)]
        + [pltpu.SemaphoreType.DMA] * 5
    )
    return pl.kernel(
        _body,
        out_type=jax.ShapeDtypeStruct((P * C,), jnp.float32),
        mesh=mesh,
        name="sc_bilinear_sampler",
        compiler_params=pltpu.CompilerParams(use_tc_tiling_on_sc=False),
        scratch_types=scratch,
    )(u2, gx, gy)


def kernel(U, grid):
    u2 = U.reshape(P, C)
    gx = grid[..., 0].reshape(P)
    gy = grid[..., 1].reshape(P)
    return _sample(u2, gx, gy).reshape(B, H, W, C)


# speedup vs baseline: 1.5730x; 1.5730x over previous
import jax
import jax.numpy as jnp
from jax import lax
from jax.experimental import pallas as pl
from jax.experimental.pallas import tpu as pltpu
from jax.experimental.pallas import tpu_sc as plsc

B, H, W, C = 4, 512, 512, 96
P = B * H * W
NC, NS, L = 2, 16, 16
NW = NC * NS
PPW = P // NW
CHUNK = 128
NCHUNK = PPW // CHUNK
NVEC = C // L
NBUF = 2


def _body(u_hbm, gx_hbm, gy_hbm, out_hbm, *scr):
    gx_v = scr[0:2]
    gy_v = scr[2:4]
    rows = (scr[4:8], scr[8:12])
    idxs = (scr[12:16], scr[16:20])
    ws = (scr[20:24], scr[24:28])
    out_v = scr[28]
    sem_grid = scr[29:31]
    sem_gat = scr[31:33]
    sem_out = scr[33]

    wid = lax.axis_index("s") * NC + lax.axis_index("c")
    base = wid * PPW
    iota = lax.iota(jnp.int32, L)
    last = NCHUNK - 1

    def grid_start(ci, j):
        cb = base + ci * CHUNK
        pltpu.make_async_copy(gx_hbm.at[pl.ds(cb, CHUNK)], gx_v[j],
                              sem_grid[j]).start()
        pltpu.make_async_copy(gy_hbm.at[pl.ds(cb, CHUNK)], gy_v[j],
                              sem_grid[j]).start()

    def grid_wait(j):
        pltpu.make_async_copy(gx_hbm.at[pl.ds(base, CHUNK)], gx_v[j],
                              sem_grid[j]).wait()
        pltpu.make_async_copy(gy_hbm.at[pl.ds(base, CHUNK)], gy_v[j],
                              sem_grid[j]).wait()

    def compute_idx(ci, j):
        cb = base + ci * CHUNK
        idxa_v, idxb_v, idxc_v, idxd_v = idxs[j]
        wa_v, wb_v, wc_v, wd_v = ws[j]
        for gi in range(CHUNK // L):
            off = gi * L
            gx = gx_v[j][pl.ds(off, L)]
            gy = gy_v[j][pl.ds(off, L)]
            px = 0.5 * ((gx + 1.0) * jnp.float32(W - 1))
            py = 0.5 * ((gy + 1.0) * jnp.float32(H - 1))
            x0 = px.astype(jnp.int32)
            y0 = py.astype(jnp.int32)
            x1 = jnp.minimum(x0 + 1, W - 1)
            y1 = jnp.minimum(y0 + 1, H - 1)
            x0f = x0.astype(jnp.float32)
            y0f = y0.astype(jnp.float32)
            x1f = x1.astype(jnp.float32)
            y1f = y1.astype(jnp.float32)

            p = cb + off + iota
            bb = (p >> 18) << 18
            ra = bb + (y0 << 9) + x0
            rb = bb + (y1 << 9) + x0
            dx01 = x1 - x0
            sl = pl.ds(off, L)
            idxa_v[sl] = ra
            idxb_v[sl] = rb
            idxc_v[sl] = ra + dx01
            idxd_v[sl] = rb + dx01

            dxa = x1f - px
            dxb = px - x0f
            dya = y1f - py
            dyb = py - y0f
            wa_v[sl] = dxa * dya
            wb_v[sl] = dxa * dyb
            wc_v[sl] = dxb * dya
            wd_v[sl] = dxb * dyb

    def gathers_start(j):
        for idx_v, rv in zip(idxs[j], rows[j]):
            pltpu.make_async_copy(u_hbm.at[idx_v], rv, sem_gat[j]).start()

    def gathers_wait(j):
        for idx_v, rv in zip(idxs[j], rows[j]):
            pltpu.make_async_copy(u_hbm.at[idx_v], rv, sem_gat[j]).wait()

    def out_start(ci):
        cb = base + ci * CHUNK
        pltpu.make_async_copy(out_v, out_hbm.at[pl.ds(cb * C, CHUNK * C)],
                              sem_out).start()

    def out_wait():
        pltpu.make_async_copy(out_v, out_hbm.at[pl.ds(base * C, CHUNK * C)],
                              sem_out).wait()

    def combine(j):
        ia_v, ib_v, ic_v, id_v = rows[j]
        wa_v, wb_v, wc_v, wd_v = ws[j]

        def pix(i, _):
            wa = wa_v[pl.ds(i, L)][0]
            wb = wb_v[pl.ds(i, L)][0]
            wc = wc_v[pl.ds(i, L)][0]
            wd = wd_v[pl.ds(i, L)][0]
            ob = i * C
            for v in range(NVEC):
                slv = pl.ds(v * L, L)
                out_v[pl.ds(ob + v * L, L)] = (
                    ia_v[i, slv] * wa + ib_v[i, slv] * wb
                    + ic_v[i, slv] * wc + id_v[i, slv] * wd)
            return _

        lax.fori_loop(0, CHUNK, pix, None)

    grid_start(0, 0)
    grid_wait(0)
    compute_idx(0, 0)
    gathers_start(0)
    grid_start(1, 1)

    def body(k, _):
        c0 = 2 * k
        c1 = c0 + 1
        grid_wait(1)
        compute_idx(c1, 1)
        gathers_start(1)
        grid_start(jnp.minimum(c0 + 2, last), 0)
        gathers_wait(0)

        @pl.when(k > 0)
        def _w0():
            out_wait()

        combine(0)
        out_start(c0)
        grid_wait(0)
        compute_idx(jnp.minimum(c0 + 2, last), 0)
        gathers_start(0)
        grid_start(jnp.minimum(c1 + 2, last), 1)
        gathers_wait(1)
        out_wait()
        combine(1)
        out_start(c1)
        return _

    lax.fori_loop(0, NCHUNK // 2, body, None)

    gathers_wait(0)
    grid_wait(1)
    out_wait()


@jax.jit
def _sample(u2, gx, gy):
    mesh = plsc.VectorSubcoreMesh(core_axis_name="c", subcore_axis_name="s",
                                  num_cores=NC, num_subcores=NS)
    vf = lambda *s: pltpu.VMEM(s, jnp.float32)
    vi = lambda *s: pltpu.VMEM(s, jnp.int32)
    scratch = (
        [vf(CHUNK)] * 2 + [vf(CHUNK)] * 2
        + [vf(CHUNK, C)] * 8
        + [vi(CHUNK)] * 8
        + [vf(CHUNK + L)] * 8
        + [vf(CHUNK * C)]
        + [pltpu.SemaphoreType.DMA] * 5
    )
    return pl.kernel(
        _body,
        out_type=jax.ShapeDtypeStruct((P * C,), jnp.float32),
        mesh=mesh,
        name="sc_bilinear_sampler",
        compiler_params=pltpu.CompilerParams(use_tc_tiling_on_sc=False),
        scratch_types=scratch,
    )(u2, gx, gy)


def kernel(U, grid):
    u2 = U.reshape(P, C)
    gx = grid[..., 0].reshape(P)
    gy = grid[..., 1].reshape(P)
    return _sample(u2, gx, gy).reshape(B, H, W, C)


# speedup vs baseline: 1.5756x; 1.0016x over previous
import jax
import jax.numpy as jnp
from jax import lax
from jax.experimental import pallas as pl
from jax.experimental.pallas import tpu as pltpu
from jax.experimental.pallas import tpu_sc as plsc

B, H, W, C = 4, 512, 512, 96
P = B * H * W
NC, NS, L = 2, 16, 16
NW = NC * NS
PPW = P // NW
CHUNK = 128
NCHUNK = PPW // CHUNK
NVEC = C // L
NBUF = 2


def _body(u_hbm, gx_hbm, gy_hbm, out_hbm, *scr):
    gx_v = scr[0:2]
    gy_v = scr[2:4]
    rows = (scr[4:8], scr[8:12])
    idxs = (scr[12:16], scr[16:20])
    ws = (scr[20:24], scr[24:28])
    out_v = scr[28]
    sem_grid = scr[29:31]
    sem_gat = scr[31:33]
    sem_out = scr[33]

    wid = lax.axis_index("s") * NC + lax.axis_index("c")
    base = wid * PPW
    iota = lax.iota(jnp.int32, L)
    last = NCHUNK - 1

    def grid_start(ci, j):
        cb = base + ci * CHUNK
        pltpu.make_async_copy(gx_hbm.at[pl.ds(cb, CHUNK)], gx_v[j],
                              sem_grid[j]).start()
        pltpu.make_async_copy(gy_hbm.at[pl.ds(cb, CHUNK)], gy_v[j],
                              sem_grid[j]).start()

    def grid_wait(j):
        pltpu.make_async_copy(gx_hbm.at[pl.ds(base, CHUNK)], gx_v[j],
                              sem_grid[j]).wait()
        pltpu.make_async_copy(gy_hbm.at[pl.ds(base, CHUNK)], gy_v[j],
                              sem_grid[j]).wait()

    def compute_idx(ci, j):
        cb = base + ci * CHUNK
        idxa_v, idxb_v, idxc_v, idxd_v = idxs[j]
        wa_v, wb_v, wc_v, wd_v = ws[j]
        for gi in range(CHUNK // L):
            off = gi * L
            gx = gx_v[j][pl.ds(off, L)]
            gy = gy_v[j][pl.ds(off, L)]
            px = 0.5 * ((gx + 1.0) * jnp.float32(W - 1))
            py = 0.5 * ((gy + 1.0) * jnp.float32(H - 1))
            x0 = px.astype(jnp.int32)
            y0 = py.astype(jnp.int32)
            x1 = jnp.minimum(x0 + 1, W - 1)
            y1 = jnp.minimum(y0 + 1, H - 1)
            x0f = x0.astype(jnp.float32)
            y0f = y0.astype(jnp.float32)
            x1f = x1.astype(jnp.float32)
            y1f = y1.astype(jnp.float32)

            p = cb + off + iota
            bb = (p >> 18) << 18
            ra = bb + (y0 << 9) + x0
            rb = bb + (y1 << 9) + x0
            dx01 = x1 - x0
            sl = pl.ds(off, L)
            idxa_v[sl] = ra
            idxb_v[sl] = rb
            idxc_v[sl] = ra + dx01
            idxd_v[sl] = rb + dx01

            dxa = x1f - px
            dxb = px - x0f
            dya = y1f - py
            dyb = py - y0f
            wa_v[sl] = dxa * dya
            wb_v[sl] = dxa * dyb
            wc_v[sl] = dxb * dya
            wd_v[sl] = dxb * dyb

    HC = CHUNK // 2

    def gathers_start(j):
        for idx_v, rv in zip(idxs[j], rows[j]):
            for h in range(2):
                pltpu.make_async_copy(u_hbm.at[idx_v.at[pl.ds(h * HC, HC)]],
                                      rv.at[pl.ds(h * HC, HC)],
                                      sem_gat[j]).start()

    def gathers_wait(j):
        for idx_v, rv in zip(idxs[j], rows[j]):
            for h in range(2):
                pltpu.make_async_copy(u_hbm.at[idx_v.at[pl.ds(h * HC, HC)]],
                                      rv.at[pl.ds(h * HC, HC)],
                                      sem_gat[j]).wait()

    def out_start(ci):
        cb = base + ci * CHUNK
        pltpu.make_async_copy(out_v, out_hbm.at[pl.ds(cb * C, CHUNK * C)],
                              sem_out).start()

    def out_wait():
        pltpu.make_async_copy(out_v, out_hbm.at[pl.ds(base * C, CHUNK * C)],
                              sem_out).wait()

    def combine(j):
        ia_v, ib_v, ic_v, id_v = rows[j]
        wa_v, wb_v, wc_v, wd_v = ws[j]

        def pix(i, _):
            wa = wa_v[pl.ds(i, L)][0]
            wb = wb_v[pl.ds(i, L)][0]
            wc = wc_v[pl.ds(i, L)][0]
            wd = wd_v[pl.ds(i, L)][0]
            ob = i * C
            for v in range(NVEC):
                slv = pl.ds(v * L, L)
                out_v[pl.ds(ob + v * L, L)] = (
                    ia_v[i, slv] * wa + ib_v[i, slv] * wb
                    + ic_v[i, slv] * wc + id_v[i, slv] * wd)
            return _

        lax.fori_loop(0, CHUNK, pix, None)

    grid_start(0, 0)
    grid_wait(0)
    compute_idx(0, 0)
    gathers_start(0)
    grid_start(1, 1)

    def body(k, _):
        c0 = 2 * k
        c1 = c0 + 1
        grid_wait(1)
        compute_idx(c1, 1)
        gathers_start(1)
        grid_start(jnp.minimum(c0 + 2, last), 0)
        gathers_wait(0)

        @pl.when(k > 0)
        def _w0():
            out_wait()

        combine(0)
        out_start(c0)
        grid_wait(0)
        compute_idx(jnp.minimum(c0 + 2, last), 0)
        gathers_start(0)
        grid_start(jnp.minimum(c1 + 2, last), 1)
        gathers_wait(1)
        out_wait()
        combine(1)
        out_start(c1)
        return _

    lax.fori_loop(0, NCHUNK // 2, body, None)

    gathers_wait(0)
    grid_wait(1)
    out_wait()


@jax.jit
def _sample(u2, gx, gy):
    mesh = plsc.VectorSubcoreMesh(core_axis_name="c", subcore_axis_name="s",
                                  num_cores=NC, num_subcores=NS)
    vf = lambda *s: pltpu.VMEM(s, jnp.float32)
    vi = lambda *s: pltpu.VMEM(s, jnp.int32)
    scratch = (
        [vf(CHUNK)] * 2 + [vf(CHUNK)] * 2
        + [vf(CHUNK, C)] * 8
        + [vi(CHUNK)] * 8
        + [vf(CHUNK + L)] * 8
        + [vf(CHUNK * C)]
        + [pltpu.SemaphoreType.DMA] * 5
    )
    return pl.kernel(
        _body,
        out_type=jax.ShapeDtypeStruct((P * C,), jnp.float32),
        mesh=mesh,
        name="sc_bilinear_sampler",
        compiler_params=pltpu.CompilerParams(use_tc_tiling_on_sc=False),
        scratch_types=scratch,
    )(u2, gx, gy)


def kernel(U, grid):
    u2 = U.reshape(P, C)
    gx = grid[..., 0].reshape(P)
    gy = grid[..., 1].reshape(P)
    return _sample(u2, gx, gy).reshape(B, H, W, C)
